# Initial kernel scaffold; baseline (speedup 1.0000x reference)
import jax
import jax.numpy as jnp
from jax.experimental import pallas as pl


def kernel(x_country, x_product, node_id_country, node_id_product, edge_src_country, edge_dst_product, edge_label_src, edge_label_dst, W_country_lin, b_country_lin, W_product_lin, b_product_lin, emb_country, emb_product, W1_c2p_l, b1_c2p, W1_c2p_r, W1_p2c_l, b1_p2c, W1_p2c_r, W2_c2p_l, b2_c2p, W2_c2p_r, W2_p2c_l, b2_p2c, W2_p2c_r):
    raise NotImplementedError("write your pallas kernel here")


# speedup vs baseline: 8.9135x; 8.9135x over previous
import functools

import jax
import jax.numpy as jnp
from jax import lax
from jax.experimental import pallas as pl
from jax.experimental.pallas import tpu as pltpu
from jax.experimental.pallas import tpu_sc as plsc

_HC = 64
_NC = 2048
_NP = 8192
_NE = 655360
_NL = 200000

_NSC = 2
_NSUB = 16
_NW = _NSC * _NSUB
_CH = 128
_EPW = _NE // _NW
_NIT = _EPW // _CH

_NLP = 200704
_PT = _NLP // _NW
_NITC = _PT // _CH

_MESH = plsc.VectorSubcoreMesh(core_axis_name="c", subcore_axis_name="s")


def _enc_body(x_ref, w_ref, b_ref, e_ref, o_ref):
    o_ref[...] = (
        jnp.dot(x_ref[...], w_ref[...], preferred_element_type=jnp.float32)
        + b_ref[...]
        + e_ref[...]
    )


def _encode(x, w, b, emb, blk):
    n, k = x.shape
    f = pl.pallas_call(
        _enc_body,
        grid=(n // blk,),
        in_specs=[
            pl.BlockSpec((blk, k), lambda i: (i, 0)),
            pl.BlockSpec((k, _HC), lambda i: (0, 0)),
            pl.BlockSpec((1, _HC), lambda i: (0, 0)),
            pl.BlockSpec((blk, _HC), lambda i: (i, 0)),
        ],
        out_specs=pl.BlockSpec((blk, _HC), lambda i: (i, 0)),
        out_shape=jax.ShapeDtypeStruct((n, _HC), jnp.float32),
    )
    return f(x, w, b.reshape(1, _HC), emb)


def _comb_body(pp, cp, hp, wlp, blp, wrp, pc, cc, hc, wlc, blc, wrc, op, oc,
               *, relu):
    def side(part, cnt, h, wl, bl, wr, out):
        agg = part[0] + part[1]
        n = cnt[0][:, 0:1] + cnt[1][:, 0:1]
        mean = agg / jnp.maximum(n, 1.0)
        r = (
            jnp.dot(mean, wl[...], preferred_element_type=jnp.float32)
            + bl[...]
            + jnp.dot(h[...], wr[...], preferred_element_type=jnp.float32)
        )
        out[...] = jnp.maximum(r, 0.0) if relu else r

    side(pp[...], cp[...], hp[...], wlp, blp, wrp, op)
    side(pc[...], cc[...], hc[...], wlc, blc, wrc, oc)


def _combine(part_p, cnt_p, h_p, wlp, blp, wrp,
             part_c, cnt_c, h_c, wlc, blc, wrc, relu):
    f = pl.pallas_call(
        functools.partial(_comb_body, relu=relu),
        out_shape=[
            jax.ShapeDtypeStruct((_NP, _HC), jnp.float32),
            jax.ShapeDtypeStruct((_NC, _HC), jnp.float32),
        ],
    )
    return f(part_p, cnt_p, h_p, wlp, blp.reshape(1, _HC), wrp,
             part_c, cnt_c, h_c, wlc, blc.reshape(1, _HC), wrc)


def _agg_body(hc_hbm, hp_hbm, src_hbm, dst_hbm, *refs, with_counts):
    if with_counts:
        (pp_hbm, pc_hbm, np_hbm, nc_hbm, src_v, dst_v, rp_v, rc_v, one_v,
         accp, accc, cntp, cntc, sem) = refs
    else:
        (pp_hbm, pc_hbm, src_v, dst_v, rp_v, rc_v,
         accp, accc, sem) = refs
    cid = lax.axis_index("c")
    sid = lax.axis_index("s")
    wid = cid * _NSUB + sid

    z16 = jnp.zeros((16,), jnp.float32)

    def zero_rows(i, _):
        for j in range(4):
            rp_v[i, pl.ds(j * 16, 16)] = z16
        if with_counts:
            one_v[i, pl.ds(0, 16)] = z16
        return 0

    lax.fori_loop(0, _CH, zero_rows, 0)

    for j in range(4):
        pltpu.sync_copy(rp_v, accp.at[pl.ds(sid * 512 + j * _CH, _CH)])
    pltpu.sync_copy(rp_v, accc.at[pl.ds(sid * _CH, _CH)])
    if with_counts:
        for j in range(4):
            pltpu.sync_copy(one_v, cntp.at[pl.ds(sid * 512 + j * _CH, _CH)])
        pltpu.sync_copy(one_v, cntc.at[pl.ds(sid * _CH, _CH)])

        o16 = jnp.ones((16,), jnp.float32)

        def one_rows(i, _):
            one_v[i, pl.ds(0, 16)] = o16
            return 0

        lax.fori_loop(0, _CH, one_rows, 0)

    plsc.subcore_barrier()

    base = wid * _EPW

    def step(i, _):
        off = base + i * _CH
        pltpu.sync_copy(src_hbm.at[pl.ds(off, _CH)], src_v)
        pltpu.sync_copy(dst_hbm.at[pl.ds(off, _CH)], dst_v)
        pltpu.async_copy(hc_hbm.at[src_v], rp_v, sem).wait()
        pltpu.sync_copy(rp_v, accp.at[dst_v], add=True)
        pltpu.async_copy(hp_hbm.at[dst_v], rc_v, sem).wait()
        pltpu.sync_copy(rc_v, accc.at[src_v], add=True)
        if with_counts:
            pltpu.sync_copy(one_v, cntp.at[dst_v], add=True)
            pltpu.sync_copy(one_v, cntc.at[src_v], add=True)
        return 0

    lax.fori_loop(0, _NIT, step, 0)
    plsc.subcore_barrier()

    pltpu.sync_copy(accp.at[pl.ds(sid * 512, 512)],
                    pp_hbm.at[cid, pl.ds(sid * 512, 512)])
    pltpu.sync_copy(accc.at[pl.ds(sid * _CH, _CH)],
                    pc_hbm.at[cid, pl.ds(sid * _CH, _CH)])
    if with_counts:
        pltpu.sync_copy(cntp.at[pl.ds(sid * 512, 512)],
                        np_hbm.at[cid, pl.ds(sid * 512, 512)])
        pltpu.sync_copy(cntc.at[pl.ds(sid * _CH, _CH)],
                        nc_hbm.at[cid, pl.ds(sid * _CH, _CH)])


def _make_agg(with_counts):
    out_type = [
        jax.ShapeDtypeStruct((_NSC, _NP, _HC), jnp.float32),
        jax.ShapeDtypeStruct((_NSC, _NC, _HC), jnp.float32),
    ]
    scratch = [
        pltpu.VMEM((_CH,), jnp.int32),
        pltpu.VMEM((_CH,), jnp.int32),
        pltpu.VMEM((_CH, _HC), jnp.float32),
        pltpu.VMEM((_CH, _HC), jnp.float32),
    ]
    if with_counts:
        out_type += [
            jax.ShapeDtypeStruct((_NSC, _NP, 16), jnp.float32),
            jax.ShapeDtypeStruct((_NSC, _NC, 16), jnp.float32),
        ]
        scratch.append(pltpu.VMEM((_CH, 16), jnp.float32))
    scratch += [
        pltpu.VMEM_SHARED((_NP, _HC), jnp.float32),
        pltpu.VMEM_SHARED((_NC, _HC), jnp.float32),
    ]
    if with_counts:
        scratch += [
            pltpu.VMEM_SHARED((_NP, 16), jnp.float32),
            pltpu.VMEM_SHARED((_NC, 16), jnp.float32),
        ]
    scratch.append(pltpu.SemaphoreType.DMA)
    return pl.kernel(
        functools.partial(_agg_body, with_counts=with_counts),
        out_type=out_type,
        mesh=_MESH,
        scratch_types=scratch,
        compiler_params=pltpu.CompilerParams(use_tc_tiling_on_sc=False),
    )


def _cls_body(c2_hbm, p2_hbm, ls_hbm, ld_hbm, out_hbm,
              ls_v, ld_v, a_v, b_v, o_v, sem):
    cid = lax.axis_index("c")
    sid = lax.axis_index("s")
    wid = cid * _NSUB + sid
    base = wid * _PT
    lane = lax.iota(jnp.int32, 16)

    def step(i, _):
        off = base + i * _CH
        pltpu.sync_copy(ls_hbm.at[pl.ds(off, _CH)], ls_v)
        pltpu.sync_copy(ld_hbm.at[pl.ds(off, _CH)], ld_v)
        pltpu.async_copy(c2_hbm.at[ls_v], a_v, sem).wait()
        pltpu.async_copy(p2_hbm.at[ld_v], b_v, sem).wait()
        for g in range(8):
            acc = jnp.zeros((16,), jnp.float32)
            for e in range(16):
                ee = g * 16 + e
                p = a_v[ee, pl.ds(0, 16)] * b_v[ee, pl.ds(0, 16)]
                p = p + a_v[ee, pl.ds(16, 16)] * b_v[ee, pl.ds(16, 16)]
                p = p + a_v[ee, pl.ds(32, 16)] * b_v[ee, pl.ds(32, 16)]
                p = p + a_v[ee, pl.ds(48, 16)] * b_v[ee, pl.ds(48, 16)]
                s = jnp.sum(p)
                acc = jnp.where(lane == e, s, acc)
            o_v[pl.ds(g * 16, 16)] = acc
        pltpu.sync_copy(o_v, out_hbm.at[wid, pl.ds(i * _CH, _CH)])
        return 0

    lax.fori_loop(0, _NITC, step, 0)


_classify = pl.kernel(
    _cls_body,
    out_type=jax.ShapeDtypeStruct((_NW, _PT), jnp.float32),
    mesh=_MESH,
    scratch_types=[
        pltpu.VMEM((_CH,), jnp.int32),
        pltpu.VMEM((_CH,), jnp.int32),
        pltpu.VMEM((_CH, _HC), jnp.float32),
        pltpu.VMEM((_CH, _HC), jnp.float32),
        pltpu.VMEM((_CH,), jnp.float32),
        pltpu.SemaphoreType.DMA,
    ],
    compiler_params=pltpu.CompilerParams(use_tc_tiling_on_sc=False,
                                         needs_layout_passes=False),
)


def kernel(x_country, x_product, node_id_country, node_id_product,
           edge_src_country, edge_dst_product, edge_label_src, edge_label_dst,
           W_country_lin, b_country_lin, W_product_lin, b_product_lin,
           emb_country, emb_product,
           W1_c2p_l, b1_c2p, W1_c2p_r, W1_p2c_l, b1_p2c, W1_p2c_r,
           W2_c2p_l, b2_c2p, W2_c2p_r, W2_p2c_l, b2_p2c, W2_p2c_r):
    h_c = _encode(x_country, W_country_lin, b_country_lin, emb_country, 256)
    h_p = _encode(x_product, W_product_lin, b_product_lin, emb_product, 256)

    agg1 = _make_agg(True)(h_c, h_p, edge_src_country, edge_dst_product)
    part_p1, part_c1, cnt_p, cnt_c = agg1
    p1, c1 = _combine(part_p1, cnt_p, h_p, W1_c2p_l, b1_c2p, W1_c2p_r,
                      part_c1, cnt_c, h_c, W1_p2c_l, b1_p2c, W1_p2c_r,
                      relu=True)

    part_p2, part_c2 = _make_agg(False)(c1, p1, edge_src_country,
                                        edge_dst_product)
    p2, c2 = _combine(part_p2, cnt_p, p1, W2_c2p_l, b2_c2p, W2_c2p_r,
                      part_c2, cnt_c, c1, W2_p2c_l, b2_p2c, W2_p2c_r,
                      relu=False)

    pad = jnp.zeros((_NLP - _NL,), jnp.int32)
    ls = jnp.concatenate([edge_label_src, pad])
    ld = jnp.concatenate([edge_label_dst, pad])
    pred = _classify(c2, p2, ls, ld)
    return pred.reshape(_NLP)[:_NL]


# speedup vs baseline: 19.1431x; 2.1477x over previous
import functools

import jax
import jax.numpy as jnp
from jax import lax
from jax.experimental import pallas as pl
from jax.experimental.pallas import tpu as pltpu
from jax.experimental.pallas import tpu_sc as plsc

_HC = 64
_NC = 2048
_NP = 8192
_NE = 655360
_NL = 200000

_NSC = 2
_NSUB = 16
_NW = _NSC * _NSUB
_CH = 128
_EPW = _NE // _NW
_NIT = _EPW // _CH
_NBUF = 2
_NBLK = _NIT // _NBUF

_NLP = 200704
_PT = _NLP // _NW
_NITC = _PT // _CH

_MESH = plsc.VectorSubcoreMesh(core_axis_name="c", subcore_axis_name="s")


def _enc_body(x_ref, w_ref, b_ref, e_ref, o_ref):
    o_ref[...] = (
        jnp.dot(x_ref[...], w_ref[...], preferred_element_type=jnp.float32)
        + b_ref[...]
        + e_ref[...]
    )


def _encode(x, w, b, emb, blk):
    n, k = x.shape
    f = pl.pallas_call(
        _enc_body,
        grid=(n // blk,),
        in_specs=[
            pl.BlockSpec((blk, k), lambda i: (i, 0)),
            pl.BlockSpec((k, _HC), lambda i: (0, 0)),
            pl.BlockSpec((1, _HC), lambda i: (0, 0)),
            pl.BlockSpec((blk, _HC), lambda i: (i, 0)),
        ],
        out_specs=pl.BlockSpec((blk, _HC), lambda i: (i, 0)),
        out_shape=jax.ShapeDtypeStruct((n, _HC), jnp.float32),
    )
    return f(x, w, b.reshape(1, _HC), emb)


def _comb_body(pp, cp, hp, wlp, blp, wrp, pc, cc, hc, wlc, blc, wrc, op, oc,
               *, relu):
    def side(part, cnt, h, wl, bl, wr, out):
        agg = part[0] + part[1]
        n = cnt[0][:, 0:1] + cnt[1][:, 0:1]
        mean = agg / jnp.maximum(n, 1.0)
        r = (
            jnp.dot(mean, wl[...], preferred_element_type=jnp.float32)
            + bl[...]
            + jnp.dot(h[...], wr[...], preferred_element_type=jnp.float32)
        )
        out[...] = jnp.maximum(r, 0.0) if relu else r

    side(pp[...], cp[...], hp[...], wlp, blp, wrp, op)
    side(pc[...], cc[...], hc[...], wlc, blc, wrc, oc)


def _combine(part_p, cnt_p, h_p, wlp, blp, wrp,
             part_c, cnt_c, h_c, wlc, blc, wrc, relu):
    f = pl.pallas_call(
        functools.partial(_comb_body, relu=relu),
        out_shape=[
            jax.ShapeDtypeStruct((_NP, _HC), jnp.float32),
            jax.ShapeDtypeStruct((_NC, _HC), jnp.float32),
        ],
    )
    return f(part_p, cnt_p, h_p, wlp, blp.reshape(1, _HC), wrp,
             part_c, cnt_c, h_c, wlc, blc.reshape(1, _HC), wrc)


def _red_body(x_ref, o_ref):
    o_ref[...] = jnp.sum(x_ref[...], axis=2)


def _reduce16(x3):
    n = x3.shape[0]
    g = 14
    blk = n // g
    f = pl.pallas_call(
        _red_body,
        grid=(g,),
        in_specs=[pl.BlockSpec((blk, _CH, 16), lambda i: (i, 0, 0))],
        out_specs=pl.BlockSpec((blk, _CH), lambda i: (i, 0)),
        out_shape=jax.ShapeDtypeStruct((n, _CH), jnp.float32),
    )
    return f(x3)


def _agg_body(hc_hbm, hp_hbm, src_hbm, dst_hbm, *refs, with_counts):
    if with_counts:
        (pp_hbm, pc_hbm, np_hbm, nc_hbm, src_v, dst_v, one_v,
         rp0, rp1, rc0, rc1,
         accp, accc, cntp, cntc, semp, semc) = refs
    else:
        (pp_hbm, pc_hbm, src_v, dst_v,
         rp0, rp1, rc0, rc1,
         accp, accc, semp, semc) = refs
    rp = (rp0, rp1)
    rc = (rc0, rc1)

    cid = lax.axis_index("c")
    sid = lax.axis_index("s")
    wid = cid * _NSUB + sid

    z16 = jnp.zeros((16,), jnp.float32)

    def zero_rows(i, _):
        for j in range(4):
            rp0[i, pl.ds(j * 16, 16)] = z16
        if with_counts:
            one_v[i, pl.ds(0, 16)] = z16
        return 0

    lax.fori_loop(0, _CH, zero_rows, 0)

    for j in range(4):
        pltpu.sync_copy(rp0, accp.at[pl.ds(sid * 512 + j * _CH, _CH)])
    pltpu.sync_copy(rp0, accc.at[pl.ds(sid * _CH, _CH)])
    if with_counts:
        for j in range(4):
            pltpu.sync_copy(one_v, cntp.at[pl.ds(sid * 512 + j * _CH, _CH)])
        pltpu.sync_copy(one_v, cntc.at[pl.ds(sid * _CH, _CH)])

        o16 = jnp.ones((16,), jnp.float32)

        def one_rows(i, _):
            one_v[i, pl.ds(0, 16)] = o16
            return 0

        lax.fori_loop(0, _CH, one_rows, 0)

    plsc.subcore_barrier()

    row0 = wid * _NIT
    pltpu.sync_copy(src_hbm.at[pl.ds(row0, _NIT)], src_v)
    pltpu.sync_copy(dst_hbm.at[pl.ds(row0, _NIT)], dst_v)

    for b in range(_NBUF):
        pltpu.async_copy(hc_hbm.at[src_v.at[b]], rp[b], semp)
        pltpu.async_copy(hp_hbm.at[dst_v.at[b]], rc[b], semc)

    def block(g, _):
        i0 = g * _NBUF
        for b in range(_NBUF):
            i = i0 + b
            j = i + _NBUF
            pltpu.make_async_copy(hc_hbm.at[src_v.at[i]], rp[b], semp).wait()
            pltpu.sync_copy(rp[b], accp.at[dst_v.at[i]], add=True)
            pltpu.async_copy(hc_hbm.at[src_v.at[j]], rp[b], semp)
            pltpu.make_async_copy(hp_hbm.at[dst_v.at[i]], rc[b], semc).wait()
            pltpu.sync_copy(rc[b], accc.at[src_v.at[i]], add=True)
            pltpu.async_copy(hp_hbm.at[dst_v.at[j]], rc[b], semc)
            if with_counts:
                pltpu.sync_copy(one_v, cntp.at[dst_v.at[i]], add=True)
                pltpu.sync_copy(one_v, cntc.at[src_v.at[i]], add=True)
        return 0

    lax.fori_loop(0, _NBLK - 1, block, 0)

    i0 = _NIT - _NBUF
    for b in range(_NBUF):
        i = i0 + b
        pltpu.make_async_copy(hc_hbm.at[src_v.at[i]], rp[b], semp).wait()
        pltpu.sync_copy(rp[b], accp.at[dst_v.at[i]], add=True)
        pltpu.make_async_copy(hp_hbm.at[dst_v.at[i]], rc[b], semc).wait()
        pltpu.sync_copy(rc[b], accc.at[src_v.at[i]], add=True)
        if with_counts:
            pltpu.sync_copy(one_v, cntp.at[dst_v.at[i]], add=True)
            pltpu.sync_copy(one_v, cntc.at[src_v.at[i]], add=True)

    plsc.subcore_barrier()

    pltpu.sync_copy(accp.at[pl.ds(sid * 512, 512)],
                    pp_hbm.at[cid, pl.ds(sid * 512, 512)])
    pltpu.sync_copy(accc.at[pl.ds(sid * _CH, _CH)],
                    pc_hbm.at[cid, pl.ds(sid * _CH, _CH)])
    if with_counts:
        pltpu.sync_copy(cntp.at[pl.ds(sid * 512, 512)],
                        np_hbm.at[cid, pl.ds(sid * 512, 512)])
        pltpu.sync_copy(cntc.at[pl.ds(sid * _CH, _CH)],
                        nc_hbm.at[cid, pl.ds(sid * _CH, _CH)])


def _make_agg(with_counts):
    out_type = [
        jax.ShapeDtypeStruct((_NSC, _NP, _HC), jnp.float32),
        jax.ShapeDtypeStruct((_NSC, _NC, _HC), jnp.float32),
    ]
    scratch = [
        pltpu.VMEM((_NIT, _CH), jnp.int32),
        pltpu.VMEM((_NIT, _CH), jnp.int32),
    ]
    if with_counts:
        out_type += [
            jax.ShapeDtypeStruct((_NSC, _NP, 16), jnp.float32),
            jax.ShapeDtypeStruct((_NSC, _NC, 16), jnp.float32),
        ]
        scratch.append(pltpu.VMEM((_CH, 16), jnp.float32))
    scratch += [
        pltpu.VMEM((_CH, _HC), jnp.float32),
        pltpu.VMEM((_CH, _HC), jnp.float32),
        pltpu.VMEM((_CH, _HC), jnp.float32),
        pltpu.VMEM((_CH, _HC), jnp.float32),
        pltpu.VMEM_SHARED((_NP, _HC), jnp.float32),
        pltpu.VMEM_SHARED((_NC, _HC), jnp.float32),
    ]
    if with_counts:
        scratch += [
            pltpu.VMEM_SHARED((_NP, 16), jnp.float32),
            pltpu.VMEM_SHARED((_NC, 16), jnp.float32),
        ]
    scratch += [pltpu.SemaphoreType.DMA, pltpu.SemaphoreType.DMA]
    return pl.kernel(
        functools.partial(_agg_body, with_counts=with_counts),
        out_type=out_type,
        mesh=_MESH,
        scratch_types=scratch,
        compiler_params=pltpu.CompilerParams(use_tc_tiling_on_sc=False),
    )


def _cls_body(c2_hbm, p2_hbm, ls_hbm, ld_hbm, out_hbm,
              ls_v, ld_v, a0, a1, b0, b1, o_v, sema, semb):
    av = (a0, a1)
    bv = (b0, b1)
    cid = lax.axis_index("c")
    sid = lax.axis_index("s")
    wid = cid * _NSUB + sid
    row0 = wid * _NITC

    pltpu.sync_copy(ls_hbm.at[pl.ds(row0, _NITC)], ls_v)
    pltpu.sync_copy(ld_hbm.at[pl.ds(row0, _NITC)], ld_v)

    for b in range(_NBUF):
        pltpu.async_copy(c2_hbm.at[ls_v.at[b]], av[b], sema)
        pltpu.async_copy(p2_hbm.at[ld_v.at[b]], bv[b], semb)

    def compute_store(i, b):
        def edge(e, _):
            p = av[b][e, pl.ds(0, 16)] * bv[b][e, pl.ds(0, 16)]
            p = p + av[b][e, pl.ds(16, 16)] * bv[b][e, pl.ds(16, 16)]
            p = p + av[b][e, pl.ds(32, 16)] * bv[b][e, pl.ds(32, 16)]
            p = p + av[b][e, pl.ds(48, 16)] * bv[b][e, pl.ds(48, 16)]
            o_v[e, pl.ds(0, 16)] = p
            return 0

        lax.fori_loop(0, _CH, edge, 0)
        pltpu.sync_copy(o_v, out_hbm.at[pl.ds((wid * _NITC + i) * _CH, _CH)])

    def block(g, _):
        i0 = g * _NBUF
        for b in range(_NBUF):
            i = i0 + b
            j = i + _NBUF
            pltpu.make_async_copy(c2_hbm.at[ls_v.at[i]], av[b], sema).wait()
            pltpu.make_async_copy(p2_hbm.at[ld_v.at[i]], bv[b], semb).wait()
            compute_store(i, b)
            pltpu.async_copy(c2_hbm.at[ls_v.at[j]], av[b], sema)
            pltpu.async_copy(p2_hbm.at[ld_v.at[j]], bv[b], semb)
        return 0

    nfull = (_NITC - 3) // _NBUF
    lax.fori_loop(0, nfull, block, 0)

    i = _NITC - 3
    pltpu.make_async_copy(c2_hbm.at[ls_v.at[i]], av[0], sema).wait()
    pltpu.make_async_copy(p2_hbm.at[ld_v.at[i]], bv[0], semb).wait()
    compute_store(i, 0)
    pltpu.async_copy(c2_hbm.at[ls_v.at[i + 2]], av[0], sema)
    pltpu.async_copy(p2_hbm.at[ld_v.at[i + 2]], bv[0], semb)

    i = _NITC - 2
    pltpu.make_async_copy(c2_hbm.at[ls_v.at[i]], av[1], sema).wait()
    pltpu.make_async_copy(p2_hbm.at[ld_v.at[i]], bv[1], semb).wait()
    compute_store(i, 1)

    i = _NITC - 1
    pltpu.make_async_copy(c2_hbm.at[ls_v.at[i]], av[0], sema).wait()
    pltpu.make_async_copy(p2_hbm.at[ld_v.at[i]], bv[0], semb).wait()
    compute_store(i, 0)


def kernel(x_country, x_product, node_id_country, node_id_product,
           edge_src_country, edge_dst_product, edge_label_src, edge_label_dst,
           W_country_lin, b_country_lin, W_product_lin, b_product_lin,
           emb_country, emb_product,
           W1_c2p_l, b1_c2p, W1_c2p_r, W1_p2c_l, b1_p2c, W1_p2c_r,
           W2_c2p_l, b2_c2p, W2_c2p_r, W2_p2c_l, b2_p2c, W2_p2c_r):
    h_c = _encode(x_country, W_country_lin, b_country_lin, emb_country, 256)
    h_p = _encode(x_product, W_product_lin, b_product_lin, emb_product, 256)

    src2d = edge_src_country.reshape(_NW * _NIT, _CH)
    dst2d = edge_dst_product.reshape(_NW * _NIT, _CH)

    agg1 = _make_agg(True)(h_c, h_p, src2d, dst2d)
    part_p1, part_c1, cnt_p, cnt_c = agg1
    p1, c1 = _combine(part_p1, cnt_p, h_p, W1_c2p_l, b1_c2p, W1_c2p_r,
                      part_c1, cnt_c, h_c, W1_p2c_l, b1_p2c, W1_p2c_r,
                      relu=True)

    part_p2, part_c2 = _make_agg(False)(c1, p1, src2d, dst2d)
    p2, c2 = _combine(part_p2, cnt_p, p1, W2_c2p_l, b2_c2p, W2_c2p_r,
                      part_c2, cnt_c, c1, W2_p2c_l, b2_p2c, W2_p2c_r,
                      relu=False)

    pad = jnp.zeros((_NLP - _NL,), jnp.int32)
    ls = jnp.concatenate([edge_label_src, pad]).reshape(_NW * _NITC, _CH)
    ld = jnp.concatenate([edge_label_dst, pad]).reshape(_NW * _NITC, _CH)

    classify = pl.kernel(
        _cls_body,
        out_type=jax.ShapeDtypeStruct((_NLP, 16), jnp.float32),
        mesh=_MESH,
        scratch_types=[
            pltpu.VMEM((_NITC, _CH), jnp.int32),
            pltpu.VMEM((_NITC, _CH), jnp.int32),
            pltpu.VMEM((_CH, _HC), jnp.float32),
            pltpu.VMEM((_CH, _HC), jnp.float32),
            pltpu.VMEM((_CH, _HC), jnp.float32),
            pltpu.VMEM((_CH, _HC), jnp.float32),
            pltpu.VMEM((_CH, 16), jnp.float32),
            pltpu.SemaphoreType.DMA,
            pltpu.SemaphoreType.DMA,
        ],
        compiler_params=pltpu.CompilerParams(use_tc_tiling_on_sc=False),
    )
    partials = classify(c2, p2, ls, ld)
    pred = _reduce16(partials.reshape(_NLP // _CH, _CH, 16))
    return pred.reshape(_NLP)[:_NL]


# speedup vs baseline: 21.9991x; 1.1492x over previous
import functools

import jax
import jax.numpy as jnp
from jax import lax
from jax.experimental import pallas as pl
from jax.experimental.pallas import tpu as pltpu
from jax.experimental.pallas import tpu_sc as plsc

_HC = 64
_NC = 2048
_NP = 8192
_NE = 655360
_NL = 200000

_NSC = 2
_NSUB = 16
_NW = _NSC * _NSUB
_CH = 128
_EPW = _NE // _NW
_NIT = _EPW // _CH
_RING = 2
_NBLK = _NIT // _RING
_NBUF = 2

_NLP = 200704
_PT = _NLP // _NW
_NITC = _PT // _CH

_MESH = plsc.VectorSubcoreMesh(core_axis_name="c", subcore_axis_name="s")


def _enc_body(x_ref, w_ref, b_ref, e_ref, o_ref):
    o_ref[...] = (
        jnp.dot(x_ref[...], w_ref[...], preferred_element_type=jnp.float32)
        + b_ref[...]
        + e_ref[...]
    )


def _encode(x, w, b, emb, blk):
    n, k = x.shape
    f = pl.pallas_call(
        _enc_body,
        grid=(n // blk,),
        in_specs=[
            pl.BlockSpec((blk, k), lambda i: (i, 0)),
            pl.BlockSpec((k, _HC), lambda i: (0, 0)),
            pl.BlockSpec((1, _HC), lambda i: (0, 0)),
            pl.BlockSpec((blk, _HC), lambda i: (i, 0)),
        ],
        out_specs=pl.BlockSpec((blk, _HC), lambda i: (i, 0)),
        out_shape=jax.ShapeDtypeStruct((n, _HC), jnp.float32),
    )
    return f(x, w, b.reshape(1, _HC), emb)


def _comb_body(pp, cp, hp, wlp, blp, wrp, pc, cc, hc, wlc, blc, wrc, op, oc,
               *, relu):
    def side(part, cnt, h, wl, bl, wr, out):
        agg = part[0] + part[1]
        n = cnt[0][:, 0:1] + cnt[1][:, 0:1]
        mean = agg / jnp.maximum(n, 1.0)
        r = (
            jnp.dot(mean, wl[...], preferred_element_type=jnp.float32)
            + bl[...]
            + jnp.dot(h[...], wr[...], preferred_element_type=jnp.float32)
        )
        out[...] = jnp.maximum(r, 0.0) if relu else r

    side(pp[...], cp[...], hp[...], wlp, blp, wrp, op)
    side(pc[...], cc[...], hc[...], wlc, blc, wrc, oc)


def _combine(part_p, cnt_p, h_p, wlp, blp, wrp,
             part_c, cnt_c, h_c, wlc, blc, wrc, relu):
    f = pl.pallas_call(
        functools.partial(_comb_body, relu=relu),
        out_shape=[
            jax.ShapeDtypeStruct((_NP, _HC), jnp.float32),
            jax.ShapeDtypeStruct((_NC, _HC), jnp.float32),
        ],
    )
    return f(part_p, cnt_p, h_p, wlp, blp.reshape(1, _HC), wrp,
             part_c, cnt_c, h_c, wlc, blc.reshape(1, _HC), wrc)


def _red_body(x_ref, m_ref, o_ref):
    o_ref[...] = jnp.dot(x_ref[...], m_ref[...],
                         preferred_element_type=jnp.float32,
                         precision=jax.lax.Precision.HIGHEST)


def _reduce16(x2, mask):
    n = x2.shape[0]
    f = pl.pallas_call(
        _red_body,
        out_shape=jax.ShapeDtypeStruct((n, _CH), jnp.float32),
    )
    return f(x2, mask)


def _agg_body(hc_hbm, hp_hbm, src_hbm, dst_hbm, *refs, with_counts):
    if with_counts:
        (pp_hbm, pc_hbm, np_hbm, nc_hbm, src_v, dst_v, one_v,
         *rest) = refs
    else:
        (pp_hbm, pc_hbm, src_v, dst_v, *rest) = refs
    rp = tuple(rest[:_RING])
    rc = tuple(rest[_RING:2 * _RING])
    if with_counts:
        accp, accc, cntp, cntc, semp, semc = rest[2 * _RING:]
    else:
        accp, accc, semp, semc = rest[2 * _RING:]
    rp0 = rp[0]

    cid = lax.axis_index("c")
    sid = lax.axis_index("s")
    wid = cid * _NSUB + sid

    z16 = jnp.zeros((16,), jnp.float32)

    def zero_rows(i, _):
        for j in range(4):
            rp0[i, pl.ds(j * 16, 16)] = z16
        if with_counts:
            one_v[i, pl.ds(0, 16)] = z16
        return 0

    lax.fori_loop(0, _CH, zero_rows, 0)

    for j in range(4):
        pltpu.sync_copy(rp0, accp.at[pl.ds(sid * 512 + j * _CH, _CH)])
    pltpu.sync_copy(rp0, accc.at[pl.ds(sid * _CH, _CH)])
    if with_counts:
        for j in range(4):
            pltpu.sync_copy(one_v, cntp.at[pl.ds(sid * 512 + j * _CH, _CH)])
        pltpu.sync_copy(one_v, cntc.at[pl.ds(sid * _CH, _CH)])

        o16 = jnp.ones((16,), jnp.float32)

        def one_rows(i, _):
            one_v[i, pl.ds(0, 16)] = o16
            return 0

        lax.fori_loop(0, _CH, one_rows, 0)

    plsc.subcore_barrier()

    row0 = wid * _NIT
    pltpu.sync_copy(src_hbm.at[pl.ds(row0, _NIT)], src_v)
    pltpu.sync_copy(dst_hbm.at[pl.ds(row0, _NIT)], dst_v)

    for b in range(_RING):
        pltpu.async_copy(hc_hbm.at[src_v.at[b]], rp[b], semp)
        pltpu.async_copy(hp_hbm.at[dst_v.at[b]], rc[b], semc)

    def block(g, _):
        i0 = g * _RING
        for b in range(_RING):
            i = i0 + b
            j = i + _RING
            pltpu.make_async_copy(hc_hbm.at[src_v.at[i]], rp[b], semp).wait()
            pltpu.sync_copy(rp[b], accp.at[dst_v.at[i]], add=True)
            pltpu.async_copy(hc_hbm.at[src_v.at[j]], rp[b], semp)
            pltpu.make_async_copy(hp_hbm.at[dst_v.at[i]], rc[b], semc).wait()
            pltpu.sync_copy(rc[b], accc.at[src_v.at[i]], add=True)
            pltpu.async_copy(hp_hbm.at[dst_v.at[j]], rc[b], semc)
            if with_counts:
                pltpu.sync_copy(one_v, cntp.at[dst_v.at[i]], add=True)
                pltpu.sync_copy(one_v, cntc.at[src_v.at[i]], add=True)
        return 0

    lax.fori_loop(0, _NBLK - 1, block, 0)

    i0 = _NIT - _RING
    for b in range(_RING):
        i = i0 + b
        pltpu.make_async_copy(hc_hbm.at[src_v.at[i]], rp[b], semp).wait()
        pltpu.sync_copy(rp[b], accp.at[dst_v.at[i]], add=True)
        pltpu.make_async_copy(hp_hbm.at[dst_v.at[i]], rc[b], semc).wait()
        pltpu.sync_copy(rc[b], accc.at[src_v.at[i]], add=True)
        if with_counts:
            pltpu.sync_copy(one_v, cntp.at[dst_v.at[i]], add=True)
            pltpu.sync_copy(one_v, cntc.at[src_v.at[i]], add=True)

    plsc.subcore_barrier()

    pltpu.sync_copy(accp.at[pl.ds(sid * 512, 512)],
                    pp_hbm.at[cid, pl.ds(sid * 512, 512)])
    pltpu.sync_copy(accc.at[pl.ds(sid * _CH, _CH)],
                    pc_hbm.at[cid, pl.ds(sid * _CH, _CH)])
    if with_counts:
        pltpu.sync_copy(cntp.at[pl.ds(sid * 512, 512)],
                        np_hbm.at[cid, pl.ds(sid * 512, 512)])
        pltpu.sync_copy(cntc.at[pl.ds(sid * _CH, _CH)],
                        nc_hbm.at[cid, pl.ds(sid * _CH, _CH)])


def _make_agg(with_counts):
    out_type = [
        jax.ShapeDtypeStruct((_NSC, _NP, _HC), jnp.float32),
        jax.ShapeDtypeStruct((_NSC, _NC, _HC), jnp.float32),
    ]
    scratch = [
        pltpu.VMEM((_NIT, _CH), jnp.int32),
        pltpu.VMEM((_NIT, _CH), jnp.int32),
    ]
    if with_counts:
        out_type += [
            jax.ShapeDtypeStruct((_NSC, _NP, 16), jnp.float32),
            jax.ShapeDtypeStruct((_NSC, _NC, 16), jnp.float32),
        ]
        scratch.append(pltpu.VMEM((_CH, 16), jnp.float32))
    scratch += [pltpu.VMEM((_CH, _HC), jnp.float32)
                for _ in range(2 * _RING)]
    scratch += [
        pltpu.VMEM_SHARED((_NP, _HC), jnp.float32),
        pltpu.VMEM_SHARED((_NC, _HC), jnp.float32),
    ]
    if with_counts:
        scratch += [
            pltpu.VMEM_SHARED((_NP, 16), jnp.float32),
            pltpu.VMEM_SHARED((_NC, 16), jnp.float32),
        ]
    scratch += [pltpu.SemaphoreType.DMA, pltpu.SemaphoreType.DMA]
    return pl.kernel(
        functools.partial(_agg_body, with_counts=with_counts),
        out_type=out_type,
        mesh=_MESH,
        scratch_types=scratch,
        compiler_params=pltpu.CompilerParams(use_tc_tiling_on_sc=False),
    )


def _cls_body(c2_hbm, p2_hbm, ls_hbm, ld_hbm, out_hbm,
              ls_v, ld_v, a0, a1, b0, b1, o_v, sema, semb):
    av = (a0, a1)
    bv = (b0, b1)
    cid = lax.axis_index("c")
    sid = lax.axis_index("s")
    wid = cid * _NSUB + sid
    row0 = wid * _NITC

    pltpu.sync_copy(ls_hbm.at[pl.ds(row0, _NITC)], ls_v)
    pltpu.sync_copy(ld_hbm.at[pl.ds(row0, _NITC)], ld_v)

    for b in range(_NBUF):
        pltpu.async_copy(c2_hbm.at[ls_v.at[b]], av[b], sema)
        pltpu.async_copy(p2_hbm.at[ld_v.at[b]], bv[b], semb)

    def compute_store(i, b):
        def edge(e, _):
            p = av[b][e, pl.ds(0, 16)] * bv[b][e, pl.ds(0, 16)]
            p = p + av[b][e, pl.ds(16, 16)] * bv[b][e, pl.ds(16, 16)]
            p = p + av[b][e, pl.ds(32, 16)] * bv[b][e, pl.ds(32, 16)]
            p = p + av[b][e, pl.ds(48, 16)] * bv[b][e, pl.ds(48, 16)]
            o_v[pl.ds(e * 16, 16)] = p
            return 0

        lax.fori_loop(0, _CH, edge, 0)
        pltpu.sync_copy(o_v, out_hbm.at[wid * _NITC + i])

    def block(g, _):
        i0 = g * _NBUF
        for b in range(_NBUF):
            i = i0 + b
            j = i + _NBUF
            pltpu.make_async_copy(c2_hbm.at[ls_v.at[i]], av[b], sema).wait()
            pltpu.make_async_copy(p2_hbm.at[ld_v.at[i]], bv[b], semb).wait()
            compute_store(i, b)
            pltpu.async_copy(c2_hbm.at[ls_v.at[j]], av[b], sema)
            pltpu.async_copy(p2_hbm.at[ld_v.at[j]], bv[b], semb)
        return 0

    nfull = (_NITC - 3) // _NBUF
    lax.fori_loop(0, nfull, block, 0)

    i = _NITC - 3
    pltpu.make_async_copy(c2_hbm.at[ls_v.at[i]], av[0], sema).wait()
    pltpu.make_async_copy(p2_hbm.at[ld_v.at[i]], bv[0], semb).wait()
    compute_store(i, 0)
    pltpu.async_copy(c2_hbm.at[ls_v.at[i + 2]], av[0], sema)
    pltpu.async_copy(p2_hbm.at[ld_v.at[i + 2]], bv[0], semb)

    i = _NITC - 2
    pltpu.make_async_copy(c2_hbm.at[ls_v.at[i]], av[1], sema).wait()
    pltpu.make_async_copy(p2_hbm.at[ld_v.at[i]], bv[1], semb).wait()
    compute_store(i, 1)

    i = _NITC - 1
    pltpu.make_async_copy(c2_hbm.at[ls_v.at[i]], av[0], sema).wait()
    pltpu.make_async_copy(p2_hbm.at[ld_v.at[i]], bv[0], semb).wait()
    compute_store(i, 0)


def kernel(x_country, x_product, node_id_country, node_id_product,
           edge_src_country, edge_dst_product, edge_label_src, edge_label_dst,
           W_country_lin, b_country_lin, W_product_lin, b_product_lin,
           emb_country, emb_product,
           W1_c2p_l, b1_c2p, W1_c2p_r, W1_p2c_l, b1_p2c, W1_p2c_r,
           W2_c2p_l, b2_c2p, W2_c2p_r, W2_p2c_l, b2_p2c, W2_p2c_r):
    h_c = _encode(x_country, W_country_lin, b_country_lin, emb_country, 256)
    h_p = _encode(x_product, W_product_lin, b_product_lin, emb_product, 256)

    src2d = edge_src_country.reshape(_NW * _NIT, _CH)
    dst2d = edge_dst_product.reshape(_NW * _NIT, _CH)

    agg1 = _make_agg(True)(h_c, h_p, src2d, dst2d)
    part_p1, part_c1, cnt_p, cnt_c = agg1
    p1, c1 = _combine(part_p1, cnt_p, h_p, W1_c2p_l, b1_c2p, W1_c2p_r,
                      part_c1, cnt_c, h_c, W1_p2c_l, b1_p2c, W1_p2c_r,
                      relu=True)

    part_p2, part_c2 = _make_agg(False)(c1, p1, src2d, dst2d)
    p2, c2 = _combine(part_p2, cnt_p, p1, W2_c2p_l, b2_c2p, W2_c2p_r,
                      part_c2, cnt_c, c1, W2_p2c_l, b2_p2c, W2_p2c_r,
                      relu=False)

    pad = jnp.zeros((_NLP - _NL,), jnp.int32)
    ls = jnp.concatenate([edge_label_src, pad]).reshape(_NW * _NITC, _CH)
    ld = jnp.concatenate([edge_label_dst, pad]).reshape(_NW * _NITC, _CH)

    classify = pl.kernel(
        _cls_body,
        out_type=jax.ShapeDtypeStruct((_NLP // _CH, _CH * 16), jnp.float32),
        mesh=_MESH,
        scratch_types=[
            pltpu.VMEM((_NITC, _CH), jnp.int32),
            pltpu.VMEM((_NITC, _CH), jnp.int32),
            pltpu.VMEM((_CH, _HC), jnp.float32),
            pltpu.VMEM((_CH, _HC), jnp.float32),
            pltpu.VMEM((_CH, _HC), jnp.float32),
            pltpu.VMEM((_CH, _HC), jnp.float32),
            pltpu.VMEM((_CH * 16,), jnp.float32),
            pltpu.SemaphoreType.DMA,
            pltpu.SemaphoreType.DMA,
        ],
        compiler_params=pltpu.CompilerParams(use_tc_tiling_on_sc=False),
    )
    partials = classify(c2, p2, ls, ld)
    mask = (jnp.arange(_CH * 16)[:, None] // 16
            == jnp.arange(_CH)[None, :]).astype(jnp.float32)
    pred = _reduce16(partials, mask)
    return pred.reshape(_NLP)[:_NL]
